# speedup vs baseline: 11.4518x; 11.4518x over previous
import functools

import jax
import jax.numpy as jnp
from jax import lax
from jax.experimental import pallas as pl
from jax.experimental.pallas import tpu as pltpu
from jax.experimental.pallas import tpu_sc as plsc

SEQ = 200
BATCH = 4096
EMBED = 128
NC = 2
NS = 16
NW = NC * NS
BPW = BATCH // NW
LANES = 16
INV_SEQ = 1.0 / SEQ


def _w2v_body(sent, table, out, idx_v, buf0, buf1, sidx, acc, sem0, sem1):
    c = lax.axis_index("c")
    s = lax.axis_index("s")
    wid = s * NC + c
    base = wid * BPW

    pltpu.sync_copy(sent.at[:, pl.ds(base, BPW)], idx_v)

    for ch in range(BPW // LANES):
        sidx[pl.ds(ch * LANES, LANES)] = (
            base + ch * LANES + lax.iota(jnp.int32, LANES)
        )

    h0 = pltpu.async_copy(table.at[idx_v.at[0]], buf0, sem0)
    h1 = pltpu.async_copy(table.at[idx_v.at[1]], buf1, sem1)

    h0.wait()
    pltpu.sync_copy(buf0, acc.at[pl.ds(base, BPW)])
    pltpu.async_copy(table.at[idx_v.at[2]], buf0, sem0)

    h1.wait()
    pltpu.sync_copy(buf1, acc.at[sidx], add=True)
    pltpu.async_copy(table.at[idx_v.at[3]], buf1, sem1)

    def gbody(g, carry):
        for bsel in range(2):
            l = 2 * g + 2 + bsel
            buf = buf0 if bsel == 0 else buf1
            sem = sem0 if bsel == 0 else sem1
            pltpu.make_async_copy(table.at[idx_v.at[l]], buf, sem).wait()
            pltpu.sync_copy(buf, acc.at[sidx], add=True)
            pltpu.async_copy(table.at[idx_v.at[l + 2]], buf, sem)
        return carry

    lax.fori_loop(0, (SEQ - 4) // 2, gbody, 0)

    pltpu.make_async_copy(table.at[idx_v.at[SEQ - 2]], buf0, sem0).wait()
    pltpu.sync_copy(buf0, acc.at[sidx], add=True)
    pltpu.make_async_copy(table.at[idx_v.at[SEQ - 1]], buf1, sem1).wait()
    pltpu.sync_copy(buf1, acc.at[sidx], add=True)

    pltpu.sync_copy(acc.at[pl.ds(base, BPW)], buf0)

    def rbody(r, carry):
        for ch in range(EMBED // LANES):
            sl = pl.ds(ch * LANES, LANES)
            buf0[r, sl] = buf0[r, sl] * INV_SEQ
        return carry

    lax.fori_loop(0, BPW, rbody, 0)
    pltpu.sync_copy(buf0, out.at[pl.ds(base, BPW)])


@jax.jit
def kernel(sentence, table):
    sentence = sentence.astype(jnp.int32)
    mesh = plsc.VectorSubcoreMesh(
        core_axis_name="c", subcore_axis_name="s", num_cores=NC, num_subcores=NS
    )
    k = functools.partial(
        pl.kernel,
        out_type=jax.ShapeDtypeStruct((BATCH, EMBED), jnp.float32),
        mesh=mesh,
        scratch_types=[
            pltpu.VMEM((SEQ, BPW), jnp.int32),
            pltpu.VMEM((BPW, EMBED), jnp.float32),
            pltpu.VMEM((BPW, EMBED), jnp.float32),
            pltpu.VMEM((BPW,), jnp.int32),
            pltpu.VMEM_SHARED((BATCH, EMBED), jnp.float32),
            pltpu.SemaphoreType.DMA,
            pltpu.SemaphoreType.DMA,
        ],
    )(_w2v_body)
    return k(sentence, table)
